# Initial kernel scaffold; baseline (speedup 1.0000x reference)
import jax
import jax.numpy as jnp
from jax.experimental import pallas as pl


def kernel(x, edge_index, W1a, b1a, W1b, b1b, W2a, b2a, W2b, b2b):
    raise NotImplementedError("write your pallas kernel here")


# speedup vs baseline: 6.5248x; 6.5248x over previous
import functools

import jax
import jax.numpy as jnp
from jax import lax
from jax.experimental import pallas as pl
from jax.experimental.pallas import tpu as pltpu
from jax.experimental.pallas import tpu_sc as plsc

N = 10000
E = 320000
D = 128
H = 128
C = 40

NC = 2
NS = 16
NW = NC * NS
EPW = E // NW
CH = 80
NCH = EPW // CH
RPT = 624
RREM = N - NS * RPT

_MESH = plsc.VectorSubcoreMesh(
    core_axis_name="c", subcore_axis_name="s", num_cores=NC, num_subcores=NS)


@functools.partial(
    pl.kernel,
    out_type=jax.ShapeDtypeStruct((NC, N, D), jnp.float32),
    mesh=_MESH,
    scratch_types=[
        pltpu.VMEM((NCH, CH), jnp.int32),
        pltpu.VMEM((NCH, CH), jnp.int32),
        pltpu.VMEM((CH, D), jnp.float32),
        pltpu.VMEM_SHARED((N, D), jnp.float32),
        pltpu.SemaphoreType.DMA,
    ],
)
def _seg_sum(table_hbm, src_hbm, dst_hbm, out_hbm,
             src_v, dst_v, rows_v, acc_sh, sem):
    cid = lax.axis_index("c")
    sid = lax.axis_index("s")
    wid = sid * NC + cid
    row0 = sid * RPT

    pltpu.sync_copy(table_hbm.at[pl.ds(row0, RPT)], acc_sh.at[pl.ds(row0, RPT)])

    @pl.when(sid == NS - 1)
    def _():
        pltpu.sync_copy(table_hbm.at[pl.ds(NS * RPT, RREM)],
                        acc_sh.at[pl.ds(NS * RPT, RREM)])

    pltpu.sync_copy(src_hbm.at[wid], src_v)
    pltpu.sync_copy(dst_hbm.at[wid], dst_v)
    plsc.subcore_barrier()

    def chunk(i, _):
        pltpu.async_copy(table_hbm.at[src_v.at[i]], rows_v, sem).wait()
        pltpu.sync_copy(rows_v, acc_sh.at[dst_v.at[i]], add=True)
        return _

    lax.fori_loop(0, NCH, chunk, None)
    plsc.subcore_barrier()

    pltpu.sync_copy(acc_sh.at[pl.ds(row0, RPT)],
                    out_hbm.at[cid, pl.ds(row0, RPT)])

    @pl.when(sid == NS - 1)
    def _():
        pltpu.sync_copy(acc_sh.at[pl.ds(NS * RPT, RREM)],
                        out_hbm.at[cid, pl.ds(NS * RPT, RREM)])


_SQRT_HALF = 0.7071067811865476


def _mlp1_body(p_ref, t_ref, wa_ref, ba_ref, wb_ref, bb_ref, h0_ref, h_ref):
    u = p_ref[0] + p_ref[1] - t_ref[...]
    z = jnp.dot(u, wa_ref[...], preferred_element_type=jnp.float32) + ba_ref[...]
    g = 0.5 * z * (1.0 + lax.erf(z * _SQRT_HALF))
    h0 = jnp.dot(g, wb_ref[...], preferred_element_type=jnp.float32) + bb_ref[...]
    h0_ref[...] = h0
    h_ref[...] = jnp.maximum(h0, 0.0)


def _mlp2_body(p_ref, t_ref, wa_ref, ba_ref, wb_ref, bb_ref, h2_ref):
    u = p_ref[0] + p_ref[1] - t_ref[...]
    z = jnp.dot(u, wa_ref[...], preferred_element_type=jnp.float32) + ba_ref[...]
    g = 0.5 * z * (1.0 + lax.erf(z * _SQRT_HALF))
    h2_ref[...] = (
        jnp.dot(g, wb_ref[...], preferred_element_type=jnp.float32) + bb_ref[...])


BN = 1000


def _mlp1(p, t, wa, ba, wb, bb):
    return pl.pallas_call(
        _mlp1_body,
        grid=(N // BN,),
        in_specs=[
            pl.BlockSpec((NC, BN, D), lambda i: (0, i, 0)),
            pl.BlockSpec((BN, D), lambda i: (i, 0)),
            pl.BlockSpec((D, H), lambda i: (0, 0)),
            pl.BlockSpec((1, H), lambda i: (0, 0)),
            pl.BlockSpec((H, H), lambda i: (0, 0)),
            pl.BlockSpec((1, H), lambda i: (0, 0)),
        ],
        out_specs=[
            pl.BlockSpec((BN, H), lambda i: (i, 0)),
            pl.BlockSpec((BN, H), lambda i: (i, 0)),
        ],
        out_shape=[
            jax.ShapeDtypeStruct((N, H), jnp.float32),
            jax.ShapeDtypeStruct((N, H), jnp.float32),
        ],
    )(p, t, wa, ba.reshape(1, H), wb, bb.reshape(1, H))


def _mlp2(p, t, wa, ba, wb, bb):
    return pl.pallas_call(
        _mlp2_body,
        grid=(N // BN,),
        in_specs=[
            pl.BlockSpec((NC, BN, H), lambda i: (0, i, 0)),
            pl.BlockSpec((BN, H), lambda i: (i, 0)),
            pl.BlockSpec((H, H), lambda i: (0, 0)),
            pl.BlockSpec((1, H), lambda i: (0, 0)),
            pl.BlockSpec((H, C), lambda i: (0, 0)),
            pl.BlockSpec((1, C), lambda i: (0, 0)),
        ],
        out_specs=pl.BlockSpec((BN, C), lambda i: (i, 0)),
        out_shape=jax.ShapeDtypeStruct((N, C), jnp.float32),
    )(p, t, wa, ba.reshape(1, H), wb, bb.reshape(1, C))


def kernel(x, edge_index, W1a, b1a, W1b, b1b, W2a, b2a, W2b, b2b):
    e = edge_index.reshape(2, NW, NCH, CH)
    src, dst = e[0], e[1]
    p1 = _seg_sum(x, src, dst)
    h0, h = _mlp1(p1, x, W1a, b1a, W1b, b1b)
    p2 = _seg_sum(h, src, dst)
    h2 = _mlp2(p2, h, W2a, b2a, W2b, b2b)
    return (h2, h0)
